# baseline (speedup 1.0000x reference)
import functools

import jax
import jax.numpy as jnp
from jax import lax
from jax.experimental import pallas as pl
from jax.experimental.pallas import tpu as pltpu
from jax.experimental.pallas import tpu_sc as plsc

_N_CLASSES = 1000000
_EMB_DIM = 32
_BATCH = 16384
_LANES = 128
_NTC = 7813

_NUM_CORES = 2
_NUM_SUBCORES = 16
_NW = _NUM_CORES * _NUM_SUBCORES
_CH = 8
_NCHUNK = 31
_WSLOTS = 8


_mesh = plsc.VectorSubcoreMesh(core_axis_name="c", subcore_axis_name="s")


@functools.partial(
    pl.kernel,
    mesh=_mesh,
    out_type=jax.ShapeDtypeStruct((_BATCH, _EMB_DIM), jnp.float32),
    scratch_types=[
        pltpu.VMEM((_BATCH,), jnp.int32),
        pltpu.VMEM((_BATCH + 16,), jnp.int32),
        pltpu.VMEM((_BATCH + 16,), jnp.int32),
        pltpu.VMEM((2, _EMB_DIM, _CH * _LANES), jnp.float32),
        pltpu.VMEM((48,), jnp.int32),
        pltpu.VMEM((48,), jnp.int32),
        pltpu.VMEM((_WSLOTS, 1, _EMB_DIM), jnp.float32),
        pltpu.HBM((1, _EMB_DIM), jnp.float32),
    ] + [pltpu.SemaphoreType.DMA] * (2 + _WSLOTS),
    compiler_params=pltpu.CompilerParams(needs_layout_passes=False),
)
def _gather_kernel(idx_hbm, table_hbm, out_hbm, idx_v, my_tc, my_me,
                   slab_v, cj, cm, rowbuf, dummy_hbm, *sems):
    wid = lax.axis_index("s") * _NUM_CORES + lax.axis_index("c")
    lo = (_NTC * wid) // _NW
    hi = (_NTC * (wid + 1)) // _NW
    sem_slab = [sems[0], sems[1]]
    sem_w = sems[2:]

    pltpu.sync_copy(idx_hbm, idx_v)

    c_lo = lax.iota(jnp.int32, 16)
    c_hi = c_lo + jnp.int32(16)

    def filt(v, off):
        iv = idx_v[pl.ds(v * 16, 16)]
        tc = lax.shift_right_logical(iv, 7)
        m = jnp.logical_and(tc >= lo, tc < hi)
        me = lax.bitwise_or(
            lax.shift_left(v * 16 + c_lo, 7),
            lax.bitwise_and(iv, jnp.int32(_LANES - 1)))
        plsc.store_compressed(my_tc.at[pl.ds(off, 16)], tc, mask=m)
        plsc.store_compressed(my_me.at[pl.ds(off, 16)], me, mask=m)
        return off + plsc.all_reduce_population_count(m)[0]

    cnt = lax.fori_loop(0, _BATCH // 16, filt, jnp.int32(0))
    nv = lax.shift_right_logical(cnt + 15, 4)
    my_tc[pl.ds(nv * 16, 16)] = jnp.full((16,), -1, jnp.int32)

    for s in range(_WSLOTS):
        pltpu.async_copy(rowbuf.at[s], dummy_hbm, sem_w[s])

    def fetch(ci, buf):
        tc0 = lo + ci * _CH
        tc0f = lax.min(tc0, jnp.int32(_NTC - _CH))
        off = pl.multiple_of(tc0f * _LANES, _LANES)
        pltpu.async_copy(
            table_hbm.at[:, pl.ds(off, _CH * _LANES)],
            slab_v.at[buf], sem_slab[buf])
        return tc0f

    def process(ci, buf):
        tc0 = lo + ci * _CH
        tc0f = lax.min(tc0, jnp.int32(_NTC - _CH))
        tc1 = lax.min(tc0 + _CH, hi)

        def scan(u, carry):
            tcv = my_tc[pl.ds(u * 16, 16)]
            mev = my_me[pl.ds(u * 16, 16)]
            m = jnp.logical_and(tcv >= tc0, tcv < tc1)
            jv = (tcv - tc0f) * _LANES + lax.bitwise_and(
                mev, jnp.int32(_LANES - 1))
            plsc.store_compressed(cj.at[pl.ds(0, 16)], jv, mask=m)
            plsc.store_compressed(cm.at[pl.ds(0, 16)], mev, mask=m)
            pc = plsc.all_reduce_population_count(m)[0]
            cjv = cj[pl.ds(0, 16)]
            cmv = cm[pl.ds(0, 16)]
            for e in range(16):
                @pl.when(e < pc)
                def _():
                    j0 = cjv[e]
                    i0 = lax.shift_right_logical(cmv[e], 7)
                    slot = e % _WSLOTS
                    pltpu.make_async_copy(
                        out_hbm.at[pl.ds(0, 1)], rowbuf.at[slot],
                        sem_w[slot]).wait()
                    v0 = plsc.load_gather(
                        slab_v.at[buf], [c_lo, jnp.full((16,), j0, jnp.int32)])
                    v1 = plsc.load_gather(
                        slab_v.at[buf], [c_hi, jnp.full((16,), j0, jnp.int32)])
                    rowbuf[slot, 0, pl.ds(0, 16)] = v0
                    rowbuf[slot, 0, pl.ds(16, 16)] = v1
                    pltpu.async_copy(
                        rowbuf.at[slot], out_hbm.at[pl.ds(i0, 1)],
                        sem_w[slot])
            return carry

        lax.fori_loop(0, nv, scan, 0)

    def step(t, carry):
        for buf in range(2):
            ci = t * 2 + buf

            @pl.when(lo + ci * _CH < hi)
            def _():
                fetch(ci, buf)
        for buf in range(2):
            ci = t * 2 + buf

            @pl.when(lo + ci * _CH < hi)
            def _():
                pltpu.make_async_copy(
                    table_hbm.at[:, pl.ds(0, _CH * _LANES)],
                    slab_v.at[buf], sem_slab[buf]).wait()
                process(ci, buf)
        return carry

    lax.fori_loop(0, (_NCHUNK + 1) // 2, step, 0)

    for s in range(_WSLOTS):
        pltpu.make_async_copy(
            out_hbm.at[pl.ds(0, 1)], rowbuf.at[s], sem_w[s]).wait()


def kernel(targets, embs):
    return _gather_kernel(targets.astype(jnp.int32), embs.T)


# speedup vs baseline: 1.4696x; 1.4696x over previous
import functools

import jax
import jax.numpy as jnp
from jax import lax
from jax.experimental import pallas as pl
from jax.experimental.pallas import tpu as pltpu
from jax.experimental.pallas import tpu_sc as plsc

_N_CLASSES = 1000000
_EMB_DIM = 32
_BATCH = 16384
_LANES = 128
_NTC = 7813

_NUM_CORES = 2
_NUM_SUBCORES = 16
_NW = _NUM_CORES * _NUM_SUBCORES
_CH = 2
_NCHUNK = 123
_ACC = 640
_MYCAP = 8192


_mesh = plsc.VectorSubcoreMesh(core_axis_name="c", subcore_axis_name="s")


@functools.partial(
    pl.kernel,
    mesh=_mesh,
    out_type=jax.ShapeDtypeStruct((_BATCH, _EMB_DIM), jnp.float32),
    scratch_types=[
        pltpu.VMEM((2048,), jnp.int32),
        pltpu.VMEM((_MYCAP + 16,), jnp.int32),
        pltpu.VMEM((_MYCAP + 16,), jnp.int32),
        pltpu.VMEM((2, _EMB_DIM, _CH * _LANES), jnp.float32),
        pltpu.VMEM((48,), jnp.int32),
        pltpu.VMEM((48,), jnp.int32),
        pltpu.VMEM((_ACC + 16, _EMB_DIM), jnp.float32),
        pltpu.VMEM((_ACC + 16,), jnp.int32),
        pltpu.SMEM((8,), jnp.int32),
        pltpu.SemaphoreType.DMA,
        pltpu.SemaphoreType.DMA,
        pltpu.SemaphoreType.DMA,
    ],
    compiler_params=pltpu.CompilerParams(needs_layout_passes=False),
)
def _gather_kernel(idx_hbm, table_hbm, out_hbm, idx_v, my_tc, my_me,
                   slab_v, cj, cm, acc_rows, acc_pos, acc_n,
                   sem_a, sem_b, sem_w):
    wid = lax.axis_index("s") * _NUM_CORES + lax.axis_index("c")
    lo = (_NTC * wid) // _NW
    hi = (_NTC * (wid + 1)) // _NW
    sem_slab = [sem_a, sem_b]

    c_lo = lax.iota(jnp.int32, 16)
    c_hi = c_lo + jnp.int32(16)
    zeros16 = jnp.zeros((16,), jnp.int32)
    cj[pl.ds(0, 16)] = zeros16
    cj[pl.ds(16, 16)] = zeros16
    cm[pl.ds(0, 16)] = zeros16
    cm[pl.ds(16, 16)] = zeros16
    acc_n[0] = jnp.int32(0)

    def filt_blk(b, off):
        pltpu.sync_copy(idx_hbm.at[pl.ds(b * 2048, 2048)], idx_v)

        def filt(v, off2):
            iv = idx_v[pl.ds(v * 16, 16)]
            tc = lax.shift_right_logical(iv, 7)
            m = jnp.logical_and(tc >= lo, tc < hi)
            me = lax.bitwise_or(
                lax.shift_left(b * 2048 + v * 16 + c_lo, 7),
                lax.bitwise_and(iv, jnp.int32(_LANES - 1)))
            off2c = lax.min(off2, jnp.int32(_MYCAP))
            plsc.store_compressed(my_tc.at[pl.ds(off2c, 16)], tc, mask=m)
            plsc.store_compressed(my_me.at[pl.ds(off2c, 16)], me, mask=m)
            return off2 + plsc.all_reduce_population_count(m)[0]

        return lax.fori_loop(0, 2048 // 16, filt, off)

    cnt = lax.fori_loop(0, _BATCH // 2048, filt_blk, jnp.int32(0))
    cnt = lax.min(cnt, jnp.int32(_MYCAP))
    nv = lax.shift_right_logical(cnt + 15, 4)
    my_tc[pl.ds(cnt, 16)] = jnp.full((16,), -1, jnp.int32)

    def fetch(ci, buf):
        tc0f = lax.min(lo + ci * _CH, jnp.int32(_NTC - _CH))
        off = pl.multiple_of(tc0f * _LANES, _LANES)
        pltpu.async_copy(
            table_hbm.at[:, pl.ds(off, _CH * _LANES)],
            slab_v.at[buf], sem_slab[buf])

    def process(ci, buf):
        tc0 = lo + ci * _CH
        tc0f = lax.min(tc0, jnp.int32(_NTC - _CH))
        tc1 = lax.min(tc0 + _CH, hi)

        def scan(u, carry):
            tcv = my_tc[pl.ds(u * 16, 16)]
            mev = my_me[pl.ds(u * 16, 16)]
            m = jnp.logical_and(tcv >= tc0, tcv < tc1)
            jv = (tcv - tc0f) * _LANES + lax.bitwise_and(
                mev, jnp.int32(_LANES - 1))
            plsc.store_compressed(cj.at[pl.ds(0, 16)], jv, mask=m)
            plsc.store_compressed(cm.at[pl.ds(0, 16)], mev, mask=m)
            pc = plsc.all_reduce_population_count(m)[0]
            a0 = acc_n[0]

            def extract(e, carry2):
                j0 = cj[pl.ds(e, 16)][0]
                i0 = lax.shift_right_logical(cm[pl.ds(e, 16)][0], 7)
                a = lax.min(a0 + e, jnp.int32(_ACC))
                av = jnp.full((16,), a, jnp.int32)
                v0 = plsc.load_gather(
                    slab_v.at[buf], [c_lo, jnp.full((16,), j0, jnp.int32)])
                v1 = plsc.load_gather(
                    slab_v.at[buf], [c_hi, jnp.full((16,), j0, jnp.int32)])
                plsc.store_scatter(acc_rows, [av, c_lo], v0)
                plsc.store_scatter(acc_rows, [av, c_hi], v1)
                plsc.store_scatter(
                    acc_pos, [av], jnp.full((16,), i0, jnp.int32),
                    mask=c_lo == 0)
                return carry2

            lax.fori_loop(0, pc, extract, 0)
            acc_n[0] = lax.min(a0 + pc, jnp.int32(_ACC))
            return carry

        lax.fori_loop(0, nv, scan, 0)

    def step(t, carry):
        for buf in range(2):
            ci = t * 2 + buf

            @pl.when(lo + ci * _CH < hi)
            def _():
                fetch(ci, buf)
        for buf in range(2):
            ci = t * 2 + buf

            @pl.when(lo + ci * _CH < hi)
            def _():
                pltpu.make_async_copy(
                    table_hbm.at[:, pl.ds(0, _CH * _LANES)],
                    slab_v.at[buf], sem_slab[buf]).wait()
                process(ci, buf)
        return carry

    lax.fori_loop(0, (_NCHUNK + 1) // 2, step, 0)

    total = acc_n[0]

    def wgroup(g, carry):
        pv = acc_pos[pl.ds(g * 16, 16)]
        for l in range(16):
            w = g * 16 + l

            @pl.when(w < total)
            def _():
                pltpu.async_copy(
                    acc_rows.at[pl.ds(w, 1)],
                    out_hbm.at[pl.ds(pv[l], 1)], sem_w)
        return carry

    nw = lax.shift_right_logical(total + 15, 4)
    lax.fori_loop(0, nw, wgroup, 0)

    def wdrain(w, carry):
        pltpu.make_async_copy(
            out_hbm.at[pl.ds(0, 1)], acc_rows.at[pl.ds(0, 1)], sem_w).wait()
        return carry

    lax.fori_loop(0, total, wdrain, 0)


def kernel(targets, embs):
    return _gather_kernel(targets.astype(jnp.int32), embs.T)


# speedup vs baseline: 2.0918x; 1.4234x over previous
import functools

import jax
import jax.numpy as jnp
from jax import lax
from jax.experimental import pallas as pl
from jax.experimental.pallas import tpu as pltpu
from jax.experimental.pallas import tpu_sc as plsc

_N_CLASSES = 1000000
_EMB_DIM = 32
_BATCH = 16384
_LANES = 128

_NUM_CORES = 2
_NUM_SUBCORES = 16
_NW = _NUM_CORES * _NUM_SUBCORES
_B_PER_W = _BATCH // _NW
_RING = 8


_mesh = plsc.VectorSubcoreMesh(core_axis_name="c", subcore_axis_name="s")


@functools.partial(
    pl.kernel,
    mesh=_mesh,
    out_type=jax.ShapeDtypeStruct((_BATCH, _EMB_DIM), jnp.float32),
    scratch_types=[
        pltpu.VMEM((_B_PER_W,), jnp.int32),
        pltpu.VMEM((_RING, _EMB_DIM, _LANES), jnp.float32),
        pltpu.VMEM((_B_PER_W, _EMB_DIM), jnp.float32),
    ] + [pltpu.SemaphoreType.DMA] * _RING,
    compiler_params=pltpu.CompilerParams(needs_layout_passes=False),
)
def _gather_kernel(idx_hbm, table_hbm, out_hbm, idx_v, cols_v, rows_v, *sems):
    wid = lax.axis_index("s") * _NUM_CORES + lax.axis_index("c")
    base = wid * _B_PER_W
    pltpu.sync_copy(idx_hbm.at[pl.ds(base, _B_PER_W)], idx_v)

    c_lo = lax.iota(jnp.int32, 16)
    c_hi = c_lo + jnp.int32(16)

    def fire(slot, t):
        tc = lax.shift_right_logical(t, 7)
        off = pl.multiple_of(tc * _LANES, _LANES)
        pltpu.async_copy(
            table_hbm.at[:, pl.ds(off, _LANES)], cols_v.at[slot], sems[slot])

    def drain_extract(slot, t, dst_i):
        pltpu.make_async_copy(
            table_hbm.at[:, pl.ds(0, _LANES)], cols_v.at[slot],
            sems[slot]).wait()
        lane = lax.bitwise_and(t, jnp.int32(_LANES - 1))
        j = jnp.full((16,), lane, jnp.int32)
        d = jnp.full((16,), dst_i, jnp.int32)
        v0 = plsc.load_gather(cols_v.at[slot], [c_lo, j])
        v1 = plsc.load_gather(cols_v.at[slot], [c_hi, j])
        plsc.store_scatter(rows_v, [d, c_lo], v0)
        plsc.store_scatter(rows_v, [d, c_hi], v1)

    def body(t, carry):
        vec = idx_v[pl.ds(t * 16, 16)]
        for half in range(2):
            for l in range(_RING):
                fire(l, vec[half * _RING + l])
            for l in range(_RING):
                drain_extract(l, vec[half * _RING + l],
                              t * 16 + half * _RING + l)
        return carry

    lax.fori_loop(0, _B_PER_W // 16, body, 0)

    pltpu.sync_copy(rows_v, out_hbm.at[pl.ds(base, _B_PER_W)])


def kernel(targets, embs):
    return _gather_kernel(targets.astype(jnp.int32), embs.T)
